# baseline (speedup 1.0000x reference)
import functools

import jax
import jax.numpy as jnp
from jax import lax
from jax.experimental import pallas as pl
from jax.experimental.pallas import tpu as pltpu
from jax.experimental.pallas import tpu_sc as plsc

B = 4096
D = 16
P1 = 64
P2 = 32
EPS = 1e-5

_NC = 2
_NS = 16
_NW = _NC * _NS
_BPW = B // _NW


def _gather_body(uid, iid, a1id, a2id, t_u, t_i, t_a1, t_a2,
                 out_u, out_i, out_a1, out_a2,
                 i0, i1, i2, i3, r0, r1, r2, r3, sem):
    wid = lax.axis_index("s") * _NC + lax.axis_index("c")
    base = wid * _BPW
    tabs = ((uid, t_u, out_u, i0, r0), (iid, t_i, out_i, i1, r1),
            (a1id, t_a1, out_a1, i2, r2), (a2id, t_a2, out_a2, i3, r3))
    for idx_hbm, _, _, idx_v, _ in tabs:
        pltpu.sync_copy(idx_hbm.at[pl.ds(base, _BPW)], idx_v)
    for _, tbl, _, idx_v, rows_v in tabs:
        def body(g, _, tbl=tbl, idx_v=idx_v, rows_v=rows_v):
            v = idx_v[pl.ds(g * 16, 16)]
            for l in range(16):
                pltpu.make_async_copy(
                    tbl.at[v[l]], rows_v.at[g * 16 + l], sem).start()
            return 0
        lax.fori_loop(0, _BPW // 16, body, 0)
    for _, tbl, _, _, rows_v in tabs:
        pltpu.make_async_copy(tbl.at[pl.ds(0, _BPW)], rows_v, sem).wait()
    for _, _, out, _, rows_v in tabs:
        pltpu.sync_copy(rows_v, out.at[pl.ds(base, _BPW)])


@functools.cache
def _gather4():
    return functools.partial(
        pl.kernel,
        mesh=plsc.VectorSubcoreMesh(core_axis_name="c", subcore_axis_name="s"),
        out_type=[jax.ShapeDtypeStruct((B, D), jnp.float32)] * 4,
        scratch_types=[pltpu.VMEM((_BPW,), jnp.int32)] * 4
        + [pltpu.VMEM((_BPW, D), jnp.float32)] * 4
        + [pltpu.SemaphoreType.DMA],
        compiler_params=pltpu.CompilerParams(use_tc_tiling_on_sc=True),
    )(_gather_body)


def _mlp_body(u_ref, i_ref, a1_ref, a2_ref,
              W1_ref, b1_ref, g1_ref, be1_ref,
              W2_ref, b2_ref, g2_ref, be2_ref,
              W3_ref, b3_ref, out_ref):
    x = jnp.concatenate(
        [u_ref[...], i_ref[...], a1_ref[...], a2_ref[...]], axis=1)
    h = jnp.dot(x, W1_ref[...].T, preferred_element_type=jnp.float32)
    h = h + b1_ref[...]
    m = jnp.mean(h, axis=0, keepdims=True)
    v = jnp.mean((h - m) ** 2, axis=0, keepdims=True)
    h = (h - m) * lax.rsqrt(v + EPS) * g1_ref[...] + be1_ref[...]
    h = jnp.maximum(h, 0.0)
    h = jnp.dot(h, W2_ref[...].T, preferred_element_type=jnp.float32)
    h = h + b2_ref[...]
    m = jnp.mean(h, axis=0, keepdims=True)
    v = jnp.mean((h - m) ** 2, axis=0, keepdims=True)
    h = (h - m) * lax.rsqrt(v + EPS) * g2_ref[...] + be2_ref[...]
    h = jnp.maximum(h, 0.0)
    out_ref[...] = (jnp.sum(h * W3_ref[...], axis=1, keepdims=True)
                    + b3_ref[...])


def _mlp(u, it, a1, a2, W1, b1, g1, be1, W2, b2, g2, be2, W3, b3):
    return pl.pallas_call(
        _mlp_body,
        out_shape=jax.ShapeDtypeStruct((B, 1), jnp.float32),
    )(u, it, a1, a2, W1, b1, g1, be1, W2, b2, g2, be2, W3, b3)


def kernel(user_id, item_id, attr1_id, attr2_id,
           emb_user, emb_item, emb_attr1, emb_attr2,
           W1, b1, g1, be1, W2, b2, g2, be2, W3, b3):
    uid = user_id.astype(jnp.int32)
    iid = item_id.astype(jnp.int32)
    a1id = attr1_id.astype(jnp.int32)
    a2id = attr2_id.astype(jnp.int32)
    u, it, a1, a2 = _gather4()(uid, iid, a1id, a2id,
                               emb_user, emb_item, emb_attr1, emb_attr2)
    return _mlp(u, it, a1, a2, W1, b1, g1, be1, W2, b2, g2, be2, W3, b3)


# speedup vs baseline: 7.2890x; 7.2890x over previous
import functools

import jax
import jax.numpy as jnp
from jax import lax
from jax.experimental import pallas as pl
from jax.experimental.pallas import tpu as pltpu
from jax.experimental.pallas import tpu_sc as plsc

B = 4096
D = 16
P1 = 64
P2 = 32
EPS = 1e-5
A = 1000

_NC = 2
_NS = 16
_NW = _NC * _NS
_BPW = B // _NW
_G = 16


def _gather_body(uid, iid, a1id, a2id, t_u, t_i, t_a1, t_a2,
                 out_u, out_i, out_a1, out_a2,
                 i0, i1, i2, i3, bufs, rows, at1, at2, sem):
    wid = lax.axis_index("s") * _NC + lax.axis_index("c")
    base = wid * _BPW
    iota = lax.iota(jnp.int32, 16)
    for idx_hbm, idx_v in ((uid, i0), (iid, i1), (a1id, i2), (a2id, i3)):
        pltpu.sync_copy(idx_hbm.at[pl.ds(base, _BPW)], idx_v)
    pltpu.sync_copy(t_a1, at1)
    pltpu.sync_copy(t_a2, at2)

    for tbl, idx_v, out in ((t_u, i0, out_u), (t_i, i1, out_i)):
        def body(g, _, tbl=tbl, idx_v=idx_v):
            v = idx_v[pl.ds(g * _G, _G)]
            q = lax.shift_right_logical(v, 7)
            s = lax.bitwise_and(v, 127)
            for l in range(_G):
                off = pl.multiple_of(q[l] * 128, 128)
                pltpu.make_async_copy(
                    tbl.at[pl.ds(0, D), pl.ds(off, 128)],
                    bufs.at[l], sem).start()
            for l in range(_G):
                pltpu.make_async_copy(
                    tbl.at[pl.ds(0, D), pl.ds(0, 128)],
                    bufs.at[l], sem).wait()
            for l in range(_G):
                col = plsc.load_gather(
                    bufs.at[l], [iota, jnp.full((16,), s[l], jnp.int32)])
                plsc.store_scatter(
                    rows, [iota, jnp.full((16,), g * _G + l, jnp.int32)],
                    col)
            return 0
        lax.fori_loop(0, _BPW // _G, body, 0)
        pltpu.sync_copy(rows, out.at[pl.ds(0, D), pl.ds(base, _BPW)])

    for at, idx_v, out in ((at1, i2, out_a1), (at2, i3, out_a2)):
        def abody(g, _, at=at, idx_v=idx_v):
            v = idx_v[pl.ds(g * _G, _G)]
            for l in range(_G):
                col = plsc.load_gather(
                    at, [iota, jnp.full((16,), v[l], jnp.int32)])
                plsc.store_scatter(
                    rows, [iota, jnp.full((16,), g * _G + l, jnp.int32)],
                    col)
            return 0
        lax.fori_loop(0, _BPW // _G, abody, 0)
        pltpu.sync_copy(rows, out.at[pl.ds(0, D), pl.ds(base, _BPW)])


@functools.cache
def _gather4():
    return functools.partial(
        pl.kernel,
        mesh=plsc.VectorSubcoreMesh(core_axis_name="c", subcore_axis_name="s"),
        out_type=[jax.ShapeDtypeStruct((D, B), jnp.float32)] * 4,
        scratch_types=[pltpu.VMEM((_BPW,), jnp.int32)] * 4
        + [pltpu.VMEM((_G, D, 128), jnp.float32),
           pltpu.VMEM((D, _BPW), jnp.float32),
           pltpu.VMEM((D, A), jnp.float32),
           pltpu.VMEM((D, A), jnp.float32),
           pltpu.SemaphoreType.DMA],
        compiler_params=pltpu.CompilerParams(needs_layout_passes=False),
    )(_gather_body)


def _mlp_body(u_ref, i_ref, a1_ref, a2_ref,
              W1_ref, b1_ref, g1_ref, be1_ref,
              W2_ref, b2_ref, g2_ref, be2_ref,
              W3_ref, b3_ref, out_ref):
    x = jnp.concatenate(
        [u_ref[...], i_ref[...], a1_ref[...], a2_ref[...]], axis=0)
    h = jnp.dot(W1_ref[...], x, preferred_element_type=jnp.float32)
    h = h + b1_ref[...][:, None]
    m = jnp.mean(h, axis=1, keepdims=True)
    v = jnp.mean((h - m) ** 2, axis=1, keepdims=True)
    h = (h - m) * lax.rsqrt(v + EPS) * g1_ref[...][:, None]
    h = h + be1_ref[...][:, None]
    h = jnp.maximum(h, 0.0)
    h = jnp.dot(W2_ref[...], h, preferred_element_type=jnp.float32)
    h = h + b2_ref[...][:, None]
    m = jnp.mean(h, axis=1, keepdims=True)
    v = jnp.mean((h - m) ** 2, axis=1, keepdims=True)
    h = (h - m) * lax.rsqrt(v + EPS) * g2_ref[...][:, None]
    h = h + be2_ref[...][:, None]
    h = jnp.maximum(h, 0.0)
    out_ref[...] = (jnp.dot(W3_ref[...], h,
                            preferred_element_type=jnp.float32)
                    + b3_ref[...][:, None])


def _mlp(u, it, a1, a2, W1, b1, g1, be1, W2, b2, g2, be2, W3, b3):
    return pl.pallas_call(
        _mlp_body,
        out_shape=jax.ShapeDtypeStruct((1, B), jnp.float32),
    )(u, it, a1, a2, W1, b1, g1, be1, W2, b2, g2, be2, W3, b3)


def kernel(user_id, item_id, attr1_id, attr2_id,
           emb_user, emb_item, emb_attr1, emb_attr2,
           W1, b1, g1, be1, W2, b2, g2, be2, W3, b3):
    uid = user_id.astype(jnp.int32)
    iid = item_id.astype(jnp.int32)
    a1id = attr1_id.astype(jnp.int32)
    a2id = attr2_id.astype(jnp.int32)
    u, it, a1, a2 = _gather4()(uid, iid, a1id, a2id,
                               emb_user.T, emb_item.T,
                               emb_attr1.T, emb_attr2.T)
    out = _mlp(u, it, a1, a2, W1, b1, g1, be1, W2, b2, g2, be2, W3, b3)
    return out.reshape(B, 1)


# speedup vs baseline: 8.7363x; 1.1986x over previous
import functools

import jax
import jax.numpy as jnp
from jax import lax
from jax.experimental import pallas as pl
from jax.experimental.pallas import tpu as pltpu
from jax.experimental.pallas import tpu_sc as plsc

B = 4096
D = 16
P1 = 64
P2 = 32
EPS = 1e-5
A = 1000

_NC = 2
_NS = 16
_NW = _NC * _NS
_BPW = B // _NW
_G = 16
_NG = _BPW // _G


def _gather_body(uid, iid, t_u, t_i, out_u, out_i,
                 i0, i1, bufs, rows, sem0, sem1):
    wid = lax.axis_index("s") * _NC + lax.axis_index("c")
    base = wid * _BPW
    iota = lax.iota(jnp.int32, 16)
    sems = (sem0, sem1)
    pltpu.sync_copy(uid.at[pl.ds(base, _BPW)], i0)
    pltpu.sync_copy(iid.at[pl.ds(base, _BPW)], i1)

    for tbl, idx_v, out in ((t_u, i0, out_u), (t_i, i1, out_i)):
        def fetch(g, bank):
            v = idx_v[pl.ds(g * _G, _G)]
            q = lax.shift_right_logical(v, 7)
            for l in range(_G):
                off = pl.multiple_of(q[l] * 128, 128)
                pltpu.make_async_copy(
                    tbl.at[pl.ds(0, D), pl.ds(off, 128)],
                    bufs.at[bank, l], sems[bank]).start()

        def extract(g, bank):
            v = idx_v[pl.ds(g * _G, _G)]
            s = lax.bitwise_and(v, 127)
            for l in range(_G):
                pltpu.make_async_copy(
                    tbl.at[pl.ds(0, D), pl.ds(0, 128)],
                    bufs.at[bank, l], sems[bank]).wait()
            for l in range(_G):
                col = plsc.load_gather(
                    bufs.at[bank, l],
                    [iota, jnp.full((16,), s[l], jnp.int32)])
                plsc.store_scatter(
                    rows, [iota, jnp.full((16,), g * _G + l, jnp.int32)],
                    col)

        fetch(0, 0)
        for g in range(_NG):
            if g + 1 < _NG:
                fetch(g + 1, (g + 1) & 1)
            extract(g, g & 1)
        pltpu.sync_copy(rows, out.at[pl.ds(0, D), pl.ds(base, _BPW)])


@functools.cache
def _gather2():
    return functools.partial(
        pl.kernel,
        mesh=plsc.VectorSubcoreMesh(core_axis_name="c", subcore_axis_name="s"),
        out_type=[jax.ShapeDtypeStruct((D, B), jnp.float32)] * 2,
        scratch_types=[pltpu.VMEM((_BPW,), jnp.int32)] * 2
        + [pltpu.VMEM((2, _G, D, 128), jnp.float32),
           pltpu.VMEM((D, _BPW), jnp.float32),
           pltpu.SemaphoreType.DMA,
           pltpu.SemaphoreType.DMA],
        compiler_params=pltpu.CompilerParams(needs_layout_passes=False),
    )(_gather_body)


def _mlp_body(u_ref, i_ref, ta1_ref, ta2_ref, a1id_ref, a2id_ref,
              W1_ref, b1_ref, g1_ref, be1_ref,
              W2_ref, b2_ref, g2_ref, be2_ref,
              W3_ref, b3_ref, out_ref):
    rows_iota = lax.broadcasted_iota(jnp.int32, (A, B), 0)
    oh1 = jnp.where(a1id_ref[...][None, :] == rows_iota, 1.0, 0.0)
    oh2 = jnp.where(a2id_ref[...][None, :] == rows_iota, 1.0, 0.0)
    a1 = jnp.dot(ta1_ref[...], oh1, preferred_element_type=jnp.float32)
    a2 = jnp.dot(ta2_ref[...], oh2, preferred_element_type=jnp.float32)
    x = jnp.concatenate([u_ref[...], i_ref[...], a1, a2], axis=0)
    h = jnp.dot(W1_ref[...], x, preferred_element_type=jnp.float32)
    h = h + b1_ref[...][:, None]
    m = jnp.mean(h, axis=1, keepdims=True)
    v = jnp.mean((h - m) ** 2, axis=1, keepdims=True)
    h = (h - m) * lax.rsqrt(v + EPS) * g1_ref[...][:, None]
    h = h + be1_ref[...][:, None]
    h = jnp.maximum(h, 0.0)
    h = jnp.dot(W2_ref[...], h, preferred_element_type=jnp.float32)
    h = h + b2_ref[...][:, None]
    m = jnp.mean(h, axis=1, keepdims=True)
    v = jnp.mean((h - m) ** 2, axis=1, keepdims=True)
    h = (h - m) * lax.rsqrt(v + EPS) * g2_ref[...][:, None]
    h = h + be2_ref[...][:, None]
    h = jnp.maximum(h, 0.0)
    out_ref[...] = (jnp.dot(W3_ref[...], h,
                            preferred_element_type=jnp.float32)
                    + b3_ref[...][:, None])


def _mlp(u, it, ta1, ta2, a1id, a2id,
         W1, b1, g1, be1, W2, b2, g2, be2, W3, b3):
    return pl.pallas_call(
        _mlp_body,
        out_shape=jax.ShapeDtypeStruct((1, B), jnp.float32),
    )(u, it, ta1, ta2, a1id, a2id,
      W1, b1, g1, be1, W2, b2, g2, be2, W3, b3)


def kernel(user_id, item_id, attr1_id, attr2_id,
           emb_user, emb_item, emb_attr1, emb_attr2,
           W1, b1, g1, be1, W2, b2, g2, be2, W3, b3):
    uid = user_id.astype(jnp.int32)
    iid = item_id.astype(jnp.int32)
    a1id = attr1_id.astype(jnp.int32)
    a2id = attr2_id.astype(jnp.int32)
    u, it = _gather2()(uid, iid, emb_user.T, emb_item.T)
    out = _mlp(u, it, emb_attr1.T, emb_attr2.T, a1id, a2id,
               W1, b1, g1, be1, W2, b2, g2, be2, W3, b3)
    return out.reshape(B, 1)


# speedup vs baseline: 9.0370x; 1.0344x over previous
import functools

import jax
import jax.numpy as jnp
from jax import lax
from jax.experimental import pallas as pl
from jax.experimental.pallas import tpu as pltpu
from jax.experimental.pallas import tpu_sc as plsc

B = 4096
D = 16
P1 = 64
P2 = 32
EPS = 1e-5
A = 1000

_NC = 2
_NS = 16
_NW = _NC * _NS
_BPW = B // _NW
_G = 16
_NB = 3
_NG = _BPW // _G


def _gather_body(uid, iid, t_u, t_i, out_u, out_i,
                 i0, i1, bufs, rows, sem0, sem1, sem2):
    wid = lax.axis_index("s") * _NC + lax.axis_index("c")
    base = wid * _BPW
    iota = lax.iota(jnp.int32, 16)
    sems = (sem0, sem1, sem2)
    pltpu.sync_copy(uid.at[pl.ds(base, _BPW)], i0)
    pltpu.sync_copy(iid.at[pl.ds(base, _BPW)], i1)

    for tbl, idx_v, out in ((t_u, i0, out_u), (t_i, i1, out_i)):
        def fetch(g, bank):
            v = idx_v[pl.ds(g * _G, _G)]
            q = lax.shift_right_logical(v, 7)
            for l in range(_G):
                off = pl.multiple_of(q[l] * 128, 128)
                pltpu.make_async_copy(
                    tbl.at[pl.ds(0, D), pl.ds(off, 128)],
                    bufs.at[bank, l], sems[bank]).start()

        def extract(g, bank):
            v = idx_v[pl.ds(g * _G, _G)]
            s = lax.bitwise_and(v, 127)
            for l in range(_G):
                pltpu.make_async_copy(
                    tbl.at[pl.ds(0, D), pl.ds(0, 128)],
                    bufs.at[bank, l], sems[bank]).wait()
            for l in range(_G):
                col = plsc.load_gather(
                    bufs.at[bank, l],
                    [iota, jnp.full((16,), s[l], jnp.int32)])
                plsc.store_scatter(
                    rows, [iota, jnp.full((16,), g * _G + l, jnp.int32)],
                    col)

        fetch(0, 0)
        fetch(1, 1)
        for g in range(_NG):
            if g + 2 < _NG:
                fetch(g + 2, (g + 2) % _NB)
            extract(g, g % _NB)
        pltpu.sync_copy(rows, out.at[pl.ds(0, D), pl.ds(base, _BPW)])


@functools.cache
def _gather2():
    return functools.partial(
        pl.kernel,
        mesh=plsc.VectorSubcoreMesh(core_axis_name="c", subcore_axis_name="s"),
        out_type=[jax.ShapeDtypeStruct((D, B), jnp.float32)] * 2,
        scratch_types=[pltpu.VMEM((_BPW,), jnp.int32)] * 2
        + [pltpu.VMEM((_NB, _G, D, 128), jnp.float32),
           pltpu.VMEM((D, _BPW), jnp.float32),
           pltpu.SemaphoreType.DMA,
           pltpu.SemaphoreType.DMA,
           pltpu.SemaphoreType.DMA],
        compiler_params=pltpu.CompilerParams(needs_layout_passes=False),
    )(_gather_body)


def _mlp_body(u_ref, i_ref, ta1_ref, ta2_ref, a1id_ref, a2id_ref,
              W1_ref, b1_ref, g1_ref, be1_ref,
              W2_ref, b2_ref, g2_ref, be2_ref,
              W3_ref, b3_ref, out_ref):
    rows_iota = lax.broadcasted_iota(jnp.int32, (A, B), 0)
    oh1 = jnp.where(a1id_ref[...][None, :] == rows_iota, 1.0, 0.0)
    oh2 = jnp.where(a2id_ref[...][None, :] == rows_iota, 1.0, 0.0)
    a1 = jnp.dot(ta1_ref[...], oh1, preferred_element_type=jnp.float32)
    a2 = jnp.dot(ta2_ref[...], oh2, preferred_element_type=jnp.float32)
    x = jnp.concatenate([u_ref[...], i_ref[...], a1, a2], axis=0)
    h = jnp.dot(W1_ref[...], x, preferred_element_type=jnp.float32)
    h = h + b1_ref[...][:, None]
    m = jnp.mean(h, axis=1, keepdims=True)
    v = jnp.mean((h - m) ** 2, axis=1, keepdims=True)
    h = (h - m) * lax.rsqrt(v + EPS) * g1_ref[...][:, None]
    h = h + be1_ref[...][:, None]
    h = jnp.maximum(h, 0.0)
    h = jnp.dot(W2_ref[...], h, preferred_element_type=jnp.float32)
    h = h + b2_ref[...][:, None]
    m = jnp.mean(h, axis=1, keepdims=True)
    v = jnp.mean((h - m) ** 2, axis=1, keepdims=True)
    h = (h - m) * lax.rsqrt(v + EPS) * g2_ref[...][:, None]
    h = h + be2_ref[...][:, None]
    h = jnp.maximum(h, 0.0)
    out_ref[...] = (jnp.dot(W3_ref[...], h,
                            preferred_element_type=jnp.float32)
                    + b3_ref[...][:, None])


def _mlp(u, it, ta1, ta2, a1id, a2id,
         W1, b1, g1, be1, W2, b2, g2, be2, W3, b3):
    return pl.pallas_call(
        _mlp_body,
        out_shape=jax.ShapeDtypeStruct((1, B), jnp.float32),
    )(u, it, ta1, ta2, a1id, a2id,
      W1, b1, g1, be1, W2, b2, g2, be2, W3, b3)


def kernel(user_id, item_id, attr1_id, attr2_id,
           emb_user, emb_item, emb_attr1, emb_attr2,
           W1, b1, g1, be1, W2, b2, g2, be2, W3, b3):
    uid = user_id.astype(jnp.int32)
    iid = item_id.astype(jnp.int32)
    a1id = attr1_id.astype(jnp.int32)
    a2id = attr2_id.astype(jnp.int32)
    u, it = _gather2()(uid, iid, emb_user.T, emb_item.T)
    out = _mlp(u, it, emb_attr1.T, emb_attr2.T, a1id, a2id,
               W1, b1, g1, be1, W2, b2, g2, be2, W3, b3)
    return out.reshape(B, 1)
